# baseline (speedup 1.0000x reference)
import jax
import jax.numpy as jnp
from jax import lax
from jax.experimental import pallas as pl
from jax.experimental.pallas import tpu as pltpu
from jax.experimental.pallas import tpu_sc as plsc

E = 1000000
D = 64
B = 16384
NC = 2
NS = 16
NW = NC * NS
RPW = B // NW
CH = 64
NCH = RPW // CH
HALF = 250000


def _body(head_h, rel_h, tail_h, enta_h, entb_h, rel5_h, out_h,
          hidx, ridx, tidx, hrowa, hrowb, rrow, trowa, trowb, outv, sem):
    iota = lax.iota(jnp.int32, 16)
    wid = lax.axis_index("s") * NC + lax.axis_index("c")
    base = wid * RPW

    for j in range(NCH):
        cbase = base + j * CH
        pltpu.sync_copy(head_h.at[pl.ds(cbase, CH)], hidx.at[j])
        pltpu.sync_copy(rel_h.at[pl.ds(cbase, CH)], ridx.at[j])
        pltpu.sync_copy(tail_h.at[pl.ds(cbase, CH)], tidx.at[j])
        for v in range(CH // 16):
            sl = pl.ds(v * 16, 16)
            hh = hidx[j, sl] >> 1
            tt = tidx[j, sl] >> 1
            hidx[j + NCH, sl] = jnp.minimum(hh, HALF - 1)
            hidx[j + 2 * NCH, sl] = jnp.maximum(hh - HALF, 0)
            tidx[j + NCH, sl] = jnp.minimum(tt, HALF - 1)
            tidx[j + 2 * NCH, sl] = jnp.maximum(tt - HALF, 0)
            ridx[j + NCH, sl] = ridx[j, sl] >> 1

    def fire(j):
        p = j % 2
        pltpu.async_copy(enta_h.at[hidx.at[j + NCH]], hrowa.at[p], sem)
        pltpu.async_copy(entb_h.at[hidx.at[j + 2 * NCH]], hrowb.at[p], sem)
        pltpu.async_copy(enta_h.at[tidx.at[j + NCH]], trowa.at[p], sem)
        pltpu.async_copy(entb_h.at[tidx.at[j + 2 * NCH]], trowb.at[p], sem)
        pltpu.async_copy(rel5_h.at[ridx.at[j + NCH]], rrow.at[p], sem)

    def drain(j):
        p = j % 2
        pltpu.make_async_copy(enta_h.at[hidx.at[j + NCH]], hrowa.at[p], sem).wait()
        pltpu.make_async_copy(entb_h.at[hidx.at[j + 2 * NCH]], hrowb.at[p], sem).wait()
        pltpu.make_async_copy(enta_h.at[tidx.at[j + NCH]], trowa.at[p], sem).wait()
        pltpu.make_async_copy(entb_h.at[tidx.at[j + 2 * NCH]], trowb.at[p], sem).wait()
        pltpu.make_async_copy(rel5_h.at[ridx.at[j + NCH]], rrow.at[p], sem).wait()

    fire(0)
    for j in range(NCH):
        p = j % 2
        drain(j)
        if j + 1 < NCH:
            fire(j + 1)

        def group(g, _):
            outvec = jnp.zeros((16,), jnp.float32)
            for i in range(16):
                row = g * 16 + i
                he = hidx[j, pl.ds(row, 16)][0]
                te = tidx[j, pl.ds(row, 16)][0]
                hp = (he & 1) * D
                rp = (ridx[j, pl.ds(row, 16)][0] & 1) * D
                tp = (te & 1) * D
                ha = he < 2 * HALF
                ta = te < 2 * HALF
                acc = None
                for c in range(D // 16):
                    hva = hrowa[p, row, pl.ds(hp + c * 16, 16)]
                    hvb = hrowb[p, row, pl.ds(hp + c * 16, 16)]
                    tva = trowa[p, row, pl.ds(tp + c * 16, 16)]
                    tvb = trowb[p, row, pl.ds(tp + c * 16, 16)]
                    hv = jnp.where(ha, hva, hvb)
                    tv = jnp.where(ta, tva, tvb)
                    rv = rrow[p, row, pl.ds(rp + c * 16, 16)]
                    d = jnp.abs(hv + rv - tv)
                    acc = d if acc is None else acc + d
                s = jnp.sum(acc)
                outvec = jnp.where(iota == i, s, outvec)
            outv[pl.ds(j * CH + g * 16, 16)] = 0.0 - outvec
            return 0

        lax.fori_loop(0, CH // 16, group, 0)

    pltpu.sync_copy(outv, out_h.at[pl.ds(base, RPW)])


@jax.jit
def _transe_sc(head, rel, tail, ent_embedding, rel_embedding):
    mesh = plsc.VectorSubcoreMesh(core_axis_name="c", subcore_axis_name="s")
    fn = pl.kernel(
        _body,
        out_type=jax.ShapeDtypeStruct((B,), jnp.float32),
        mesh=mesh,
        compiler_params=pltpu.CompilerParams(
            needs_layout_passes=False, use_tc_tiling_on_sc=True),
        scratch_types=[
            pltpu.VMEM((3 * NCH, CH), jnp.int32),
            pltpu.VMEM((2 * NCH, CH), jnp.int32),
            pltpu.VMEM((3 * NCH, CH), jnp.int32),
            pltpu.VMEM((2, CH, 128), jnp.float32),
            pltpu.VMEM((2, CH, 128), jnp.float32),
            pltpu.VMEM((2, CH, 128), jnp.float32),
            pltpu.VMEM((2, CH, 128), jnp.float32),
            pltpu.VMEM((2, CH, 128), jnp.float32),
            pltpu.VMEM((RPW,), jnp.float32),
            pltpu.SemaphoreType.DMA,
        ],
    )
    enta = ent_embedding[:2 * HALF].reshape(HALF, 128)
    entb = ent_embedding[2 * HALF:].reshape(HALF, 128)
    rel5 = jnp.pad(rel_embedding, ((0, 24), (0, 0))).reshape(512, 128)
    return fn(head, rel, tail, enta, entb, rel5)


def kernel(head, rel, tail, ent_embedding, rel_embedding):
    return _transe_sc(head, rel, tail, ent_embedding, rel_embedding).reshape(B, 1)


# speedup vs baseline: 2.4529x; 2.4529x over previous
import functools

import jax
import jax.numpy as jnp
from jax import lax
from jax.experimental import pallas as pl
from jax.experimental.pallas import tpu as pltpu
from jax.experimental.pallas import tpu_sc as plsc

B = 16384
D = 64
NC = 2
NS = 16
NW = NC * NS
RPW = B // NW
CH = 128
NCH = RPW // CH
G = RPW // 16


def _body(head_h, rel_h, tail_h, ent_h, remb_h, out_h,
          hidx, ridx, tidx, hrow, rrow, trow, outv, sem):
    wid = lax.axis_index("s") * NC + lax.axis_index("c")
    base = wid * RPW

    pltpu.sync_copy(head_h.at[pl.ds(base, RPW)], hidx)
    pltpu.sync_copy(rel_h.at[pl.ds(base, RPW)], ridx)
    pltpu.sync_copy(tail_h.at[pl.ds(base, RPW)], tidx)

    copies = []
    for j in range(NCH):
        sl = pl.ds(j * CH, CH)
        copies.append(pltpu.async_copy(ent_h.at[hidx.at[sl]], hrow.at[sl], sem))
        copies.append(pltpu.async_copy(ent_h.at[tidx.at[sl]], trow.at[sl], sem))
        copies.append(pltpu.async_copy(remb_h.at[ridx.at[sl]], rrow.at[sl], sem))
    for c in copies:
        c.wait()

    iota = lax.iota(jnp.int32, 16)

    def group(g, carry):
        rbase = g * 16
        outvec = jnp.zeros((16,), jnp.float32)
        for i in range(16):
            row = rbase + i
            acc = None
            for c in range(D // 16):
                sl = pl.ds(c * 16, 16)
                hv = hrow[row, sl]
                rv = rrow[row, sl]
                tv = trow[row, sl]
                d = jnp.abs(hv + rv - tv)
                acc = d if acc is None else acc + d
            s = jnp.sum(acc)
            outvec = jnp.where(iota == i, s, outvec)
        outv[pl.ds(rbase, 16)] = 0.0 - outvec
        return carry

    lax.fori_loop(0, G, group, 0)
    pltpu.sync_copy(outv, out_h.at[pl.ds(base, RPW)])


@jax.jit
def _transe_sc(head, rel, tail, ent_embedding, rel_embedding):
    mesh = plsc.VectorSubcoreMesh(core_axis_name="c", subcore_axis_name="s")
    fn = pl.kernel(
        _body,
        out_type=jax.ShapeDtypeStruct((B,), jnp.float32),
        mesh=mesh,
        compiler_params=pltpu.CompilerParams(
            needs_layout_passes=False, use_tc_tiling_on_sc=False),
        scratch_types=[
            pltpu.VMEM((RPW,), jnp.int32),
            pltpu.VMEM((RPW,), jnp.int32),
            pltpu.VMEM((RPW,), jnp.int32),
            pltpu.VMEM((RPW, D), jnp.float32),
            pltpu.VMEM((RPW, D), jnp.float32),
            pltpu.VMEM((RPW, D), jnp.float32),
            pltpu.VMEM((RPW,), jnp.float32),
            pltpu.SemaphoreType.DMA,
        ],
    )
    return fn(head, rel, tail, ent_embedding, rel_embedding)


def kernel(head, rel, tail, ent_embedding, rel_embedding):
    out = _transe_sc(head, rel, tail, ent_embedding, rel_embedding)
    return out.reshape(B, 1)
